# baseline (speedup 1.0000x reference)
import functools

import jax
import jax.numpy as jnp
from jax import lax
from jax.experimental import pallas as pl
from jax.experimental.pallas import tpu as pltpu
from jax.experimental.pallas import tpu_sc as plsc

N = 10000
E = 160000
D = 256
H = 128
NC = 2
NS = 16
CHUNK = 128
E_PAD = 163840
NROWS = E_PAD // CHUNK
NCHUNK = NROWS // NS
N_PAD = 10240
NPT = N_PAD // NS
DEG_W = 128
MB = 1000

_MESH = dict(core_axis_name="c", subcore_axis_name="s", num_cores=NC,
             num_subcores=NS)


def _deg_body(edges_hbm, out_hbm, idx_v, val_v, acc, sem):
    del sem
    c = lax.axis_index("c")
    s = lax.axis_index("s")

    def _fill(v):
        def f(j, _):
            for k in range(DEG_W // 16):
                val_v[j, pl.ds(k * 16, 16)] = jnp.full((16,), v, jnp.float32)
            return 0
        lax.fori_loop(0, CHUNK, f, 0)

    _fill(0.0)
    for k in range(NPT // CHUNK):
        pltpu.sync_copy(val_v, acc.at[pl.ds(s * NPT + k * CHUNK, CHUNK)])
    _fill(1.0)
    pltpu.sync_copy(edges_hbm.at[pl.ds(c * NROWS + s * NCHUNK, NCHUNK)], idx_v)
    plsc.subcore_barrier()

    def body(j, _):
        pltpu.sync_copy(val_v, acc.at[idx_v.at[j]], add=True)
        return 0
    lax.fori_loop(0, NCHUNK, body, 0)
    plsc.subcore_barrier()
    pltpu.sync_copy(acc.at[pl.ds(s * NPT, NPT)],
                    out_hbm.at[c].at[pl.ds(s * NPT, NPT)])


@functools.partial(
    pl.kernel,
    out_type=jax.ShapeDtypeStruct((NC, N_PAD, DEG_W), jnp.float32),
    mesh=plsc.VectorSubcoreMesh(**_MESH),
    scratch_types=[
        pltpu.VMEM((NCHUNK, CHUNK), jnp.int32),
        pltpu.VMEM((CHUNK, DEG_W), jnp.float32),
        pltpu.VMEM_SHARED((N_PAD, DEG_W), jnp.float32),
        pltpu.SemaphoreType.DMA,
    ],
)
def _deg(*args):
    _deg_body(*args)


SECS = 2
SCHUNK = NCHUNK // SECS


def _agg_body(hw_hbm, srcoff_hbm, dst_hbm, out_hbm, idx_s, idx_d,
              b0, b1, acc, g0, g1, s0, s1):
    c = lax.axis_index("c")
    s = lax.axis_index("s")
    bufs = (b0, b1)
    gsem = (g0, g1)
    ssem = (s0, s1)

    def zrow(j, _):
        for k in range(H // 16):
            b0[j, pl.ds(k * 16, 16)] = jnp.zeros((16,), jnp.float32)
        return 0
    lax.fori_loop(0, CHUNK, zrow, 0)
    for k in range(NPT // CHUNK):
        pltpu.sync_copy(b0, acc.at[pl.ds(s * NPT + k * CHUNK, CHUNK)])
    plsc.subcore_barrier()

    def gather(j, b):
        pltpu.async_copy(hw_hbm.at[idx_s.at[j]], bufs[b], gsem[b])

    def gather_wait(j, b):
        pltpu.make_async_copy(hw_hbm.at[idx_s.at[j]], bufs[b],
                              gsem[b]).wait()

    def scatter(j, b):
        pltpu.async_copy(bufs[b], acc.at[idx_d.at[j]], ssem[b], add=True)

    def scatter_wait(j, b):
        pltpu.make_async_copy(bufs[b], acc.at[idx_d.at[j]], ssem[b]).wait()

    base = c * NROWS + s * NCHUNK
    for sec in range(SECS):
        pltpu.sync_copy(
            srcoff_hbm.at[pl.ds(base + sec * SCHUNK, SCHUNK)], idx_s)
        pltpu.sync_copy(
            dst_hbm.at[pl.ds(s * NCHUNK + sec * SCHUNK, SCHUNK)], idx_d)
        gather(0, 0)
        gather(1, 1)

        def group(g, _):
            for b in range(2):
                j = g * 2 + b
                gather_wait(j, b)
                scatter(j, b)
            for b in range(2):
                j = g * 2 + b
                scatter_wait(j, b)
                gather(j + 2, b)
            return 0
        lax.fori_loop(0, (SCHUNK - 2) // 2, group, 0)
        for j in range(SCHUNK - 2, SCHUNK):
            b = j % 2
            gather_wait(j, b)
            scatter(j, b)
        for j in range(SCHUNK - 2, SCHUNK):
            scatter_wait(j, j % 2)

    plsc.subcore_barrier()
    pltpu.sync_copy(acc.at[pl.ds(s * NPT, NPT)],
                    out_hbm.at[pl.ds(c * N_PAD + s * NPT, NPT)])


@functools.partial(
    pl.kernel,
    out_type=jax.ShapeDtypeStruct((NC * N_PAD, H), jnp.float32),
    mesh=plsc.VectorSubcoreMesh(**_MESH),
    scratch_types=[
        pltpu.VMEM((SCHUNK, CHUNK), jnp.int32),
        pltpu.VMEM((SCHUNK, CHUNK), jnp.int32),
        pltpu.VMEM((CHUNK, H), jnp.float32),
        pltpu.VMEM((CHUNK, H), jnp.float32),
        pltpu.VMEM_SHARED((N_PAD, H), jnp.float32),
        pltpu.SemaphoreType.DMA,
        pltpu.SemaphoreType.DMA,
        pltpu.SemaphoreType.DMA,
        pltpu.SemaphoreType.DMA,
    ],
)
def _agg(*args):
    _agg_body(*args)


def _norm(deg_col):
    return jnp.where(deg_col > 0, lax.rsqrt(deg_col), 0.0)


def _mm1_body(x_ref, w_ref, odeg_ref, out_ref):
    nout = _norm(odeg_ref[:, :1])
    hw = jnp.dot(x_ref[...], w_ref[...],
                 preferred_element_type=jnp.float32) * nout
    out_ref[0] = hw[:, :H]
    out_ref[1] = hw[:, H:]


def _mm2_body(a_ref, b_ref, w_ref, ideg_ref, odeg_ref, b1_ref, out_ref):
    nin = _norm(ideg_ref[:, :1])
    nout = _norm(odeg_ref[:, :1])
    bias = b1_ref[...]
    ha = jnp.maximum(a_ref[0] * nin + bias[:, :H], 0.0)
    hb = jnp.maximum(b_ref[0] * nin + bias[:, H:], 0.0)
    hw = (jnp.dot(ha, w_ref[:H, :], preferred_element_type=jnp.float32) +
          jnp.dot(hb, w_ref[H:, :], preferred_element_type=jnp.float32))
    hw = hw * nout
    out_ref[0] = hw[:, :H]
    out_ref[1] = hw[:, H:]


def _epi_body(a_ref, b_ref, ideg_ref, b2_ref, out_ref):
    nin = _norm(ideg_ref[:, :1])
    bias = b2_ref[...]
    out_ref[:, :H] = a_ref[0] * nin + bias[:, :H]
    out_ref[:, H:] = b_ref[0] * nin + bias[:, H:]


_GRID = (N // MB,)
_deg_spec = pl.BlockSpec((MB, DEG_W), lambda i: (i, 0))
_half_spec_a = pl.BlockSpec((1, MB, H), lambda i: (0, i, 0))
_half_spec_b = pl.BlockSpec((1, MB, H), lambda i: (1, i, 0))
_bias_spec = pl.BlockSpec((1, D), lambda i: (0, 0))
_w_spec = pl.BlockSpec((D, D), lambda i: (0, 0))
_out2_spec = pl.BlockSpec((2, MB, H), lambda i: (0, i, 0))

_mm1 = pl.pallas_call(
    _mm1_body,
    grid=_GRID,
    in_specs=[pl.BlockSpec((MB, D), lambda i: (i, 0)), _w_spec, _deg_spec],
    out_specs=_out2_spec,
    out_shape=jax.ShapeDtypeStruct((2, N_PAD, H), jnp.float32),
)

_mm2 = pl.pallas_call(
    _mm2_body,
    grid=_GRID,
    in_specs=[_half_spec_a, _half_spec_b, _w_spec, _deg_spec, _deg_spec,
              _bias_spec],
    out_specs=_out2_spec,
    out_shape=jax.ShapeDtypeStruct((2, N_PAD, H), jnp.float32),
)

_epi = pl.pallas_call(
    _epi_body,
    grid=_GRID,
    in_specs=[_half_spec_a, _half_spec_b, _deg_spec, _bias_spec],
    out_specs=pl.BlockSpec((MB, D), lambda i: (i, 0)),
    out_shape=jax.ShapeDtypeStruct((N, D), jnp.float32),
)


def kernel(features, edge_index, W1, b1, W2, b2):
    src = edge_index[0]
    dst = edge_index[1]
    npad = E_PAD - E
    pad_idx = N + jnp.arange(npad, dtype=jnp.int32) % (N_PAD - N)
    pad_src = jnp.arange(npad, dtype=jnp.int32) % N
    srcp = jnp.concatenate([src, pad_src])
    srcp = srcp.reshape(NROWS, CHUNK)
    src2 = jnp.concatenate([srcp, srcp + N_PAD], axis=0)
    dstp = jnp.concatenate([dst, pad_idx])
    dstp = dstp.reshape(NROWS, CHUNK)
    edges2 = jnp.concatenate([
        jnp.concatenate([src, pad_idx]).reshape(NROWS, CHUNK),
        jnp.concatenate([dst, pad_idx]).reshape(NROWS, CHUNK),
    ], axis=0)

    degs = _deg(edges2)
    odeg = degs[0]
    ideg = degs[1]

    hw1 = _mm1(features, W1, odeg)
    agg1 = _agg(hw1.reshape(NC * N_PAD, H), src2, dstp).reshape(NC, N_PAD, H)
    hw2 = _mm2(agg1, agg1, W2, ideg, odeg, b1.reshape(1, D))
    agg2 = _agg(hw2.reshape(NC * N_PAD, H), src2, dstp).reshape(NC, N_PAD, H)
    return _epi(agg2, agg2, ideg, b2.reshape(1, D))


# speedup vs baseline: 1.2142x; 1.2142x over previous
import functools

import jax
import jax.numpy as jnp
from jax import lax
from jax.experimental import pallas as pl
from jax.experimental.pallas import tpu as pltpu
from jax.experimental.pallas import tpu_sc as plsc

N = 10000
E = 160000
D = 256
H = 128
NC = 2
NS = 16
CHUNK = 128
E_PAD = 163840
NROWS = E_PAD // CHUNK
NCHUNK = NROWS // NS
N_PAD = 10240
NPT = N_PAD // NS
DEG_W = 128
MB = 1000

_MESH = dict(core_axis_name="c", subcore_axis_name="s", num_cores=NC,
             num_subcores=NS)


def _deg_body(edges_hbm, out_hbm, idx_v, val_v, acc, sem):
    del sem
    c = lax.axis_index("c")
    s = lax.axis_index("s")

    def _fill(v):
        def f(j, _):
            for k in range(DEG_W // 16):
                val_v[j, pl.ds(k * 16, 16)] = jnp.full((16,), v, jnp.float32)
            return 0
        lax.fori_loop(0, CHUNK, f, 0)

    _fill(0.0)
    for k in range(NPT // CHUNK):
        pltpu.sync_copy(val_v, acc.at[pl.ds(s * NPT + k * CHUNK, CHUNK)])
    _fill(1.0)
    pltpu.sync_copy(edges_hbm.at[pl.ds(c * NROWS + s * NCHUNK, NCHUNK)], idx_v)
    plsc.subcore_barrier()

    def body(j, _):
        pltpu.sync_copy(val_v, acc.at[idx_v.at[j]], add=True)
        return 0
    lax.fori_loop(0, NCHUNK, body, 0)
    plsc.subcore_barrier()
    pltpu.sync_copy(acc.at[pl.ds(s * NPT, NPT)],
                    out_hbm.at[c].at[pl.ds(s * NPT, NPT)])


@functools.partial(
    pl.kernel,
    out_type=jax.ShapeDtypeStruct((NC, N_PAD, DEG_W), jnp.float32),
    mesh=plsc.VectorSubcoreMesh(**_MESH),
    scratch_types=[
        pltpu.VMEM((NCHUNK, CHUNK), jnp.int32),
        pltpu.VMEM((CHUNK, DEG_W), jnp.float32),
        pltpu.VMEM_SHARED((N_PAD, DEG_W), jnp.float32),
        pltpu.SemaphoreType.DMA,
    ],
)
def _deg(*args):
    _deg_body(*args)


SECS = 2
SCHUNK = NCHUNK // SECS


def _agg_body(hw_hbm, srcoff_hbm, dst_hbm, out_hbm, idx_s, idx_d,
              b0, b1, acc, g0, g1):
    c = lax.axis_index("c")
    s = lax.axis_index("s")
    bufs = (b0, b1)
    gsem = (g0, g1)

    def gather(j, b):
        pltpu.async_copy(hw_hbm.at[idx_s.at[j]], bufs[b], gsem[b])

    def gather_wait(j, b):
        pltpu.make_async_copy(hw_hbm.at[idx_s.at[j]], bufs[b],
                              gsem[b]).wait()

    base = c * NROWS + s * NCHUNK
    pltpu.sync_copy(srcoff_hbm.at[pl.ds(base, SCHUNK)], idx_s)
    pltpu.sync_copy(dst_hbm.at[pl.ds(s * NCHUNK, SCHUNK)], idx_d)
    gather(0, 0)

    def zrow(j, _):
        for k in range(H // 16):
            b1[j, pl.ds(k * 16, 16)] = jnp.zeros((16,), jnp.float32)
        return 0
    lax.fori_loop(0, CHUNK, zrow, 0)
    for k in range(NPT // CHUNK):
        pltpu.sync_copy(b1, acc.at[pl.ds(s * NPT + k * CHUNK, CHUNK)])
    plsc.subcore_barrier()

    for sec in range(SECS):
        if sec > 0:
            pltpu.sync_copy(
                srcoff_hbm.at[pl.ds(base + sec * SCHUNK, SCHUNK)], idx_s)
            pltpu.sync_copy(
                dst_hbm.at[pl.ds(s * NCHUNK + sec * SCHUNK, SCHUNK)], idx_d)
            gather(0, 0)
        gather(1, 1)

        def group(g, _):
            for b in range(2):
                j = g * 2 + b
                gather_wait(j, b)
                pltpu.sync_copy(bufs[b], acc.at[idx_d.at[j]], add=True)
                gather(j + 2, b)
            return 0
        lax.fori_loop(0, (SCHUNK - 2) // 2, group, 0)
        for j in range(SCHUNK - 2, SCHUNK):
            b = j % 2
            gather_wait(j, b)
            pltpu.sync_copy(bufs[b], acc.at[idx_d.at[j]], add=True)

    plsc.subcore_barrier()
    pltpu.sync_copy(acc.at[pl.ds(s * NPT, NPT)],
                    out_hbm.at[pl.ds(c * N_PAD + s * NPT, NPT)])


@functools.partial(
    pl.kernel,
    out_type=jax.ShapeDtypeStruct((NC * N_PAD, H), jnp.float32),
    mesh=plsc.VectorSubcoreMesh(**_MESH),
    scratch_types=[
        pltpu.VMEM((SCHUNK, CHUNK), jnp.int32),
        pltpu.VMEM((SCHUNK, CHUNK), jnp.int32),
        pltpu.VMEM((CHUNK, H), jnp.float32),
        pltpu.VMEM((CHUNK, H), jnp.float32),
        pltpu.VMEM_SHARED((N_PAD, H), jnp.float32),
        pltpu.SemaphoreType.DMA,
        pltpu.SemaphoreType.DMA,
    ],
)
def _agg(*args):
    _agg_body(*args)


def _norm(deg_col):
    return jnp.where(deg_col > 0, lax.rsqrt(deg_col), 0.0)


def _mm1_body(x_ref, w_ref, odeg_ref, out_ref):
    nout = _norm(odeg_ref[:, :1])
    hw = jnp.dot(x_ref[...], w_ref[...],
                 preferred_element_type=jnp.float32) * nout
    out_ref[0] = hw[:, :H]
    out_ref[1] = hw[:, H:]


def _mm2_body(a_ref, b_ref, w_ref, ideg_ref, odeg_ref, b1_ref, out_ref):
    nin = _norm(ideg_ref[:, :1])
    nout = _norm(odeg_ref[:, :1])
    bias = b1_ref[...]
    ha = jnp.maximum(a_ref[0] * nin + bias[:, :H], 0.0)
    hb = jnp.maximum(b_ref[0] * nin + bias[:, H:], 0.0)
    hw = (jnp.dot(ha, w_ref[:H, :], preferred_element_type=jnp.float32) +
          jnp.dot(hb, w_ref[H:, :], preferred_element_type=jnp.float32))
    hw = hw * nout
    out_ref[0] = hw[:, :H]
    out_ref[1] = hw[:, H:]


def _epi_body(a_ref, b_ref, ideg_ref, b2_ref, out_ref):
    nin = _norm(ideg_ref[:, :1])
    bias = b2_ref[...]
    out_ref[:, :H] = a_ref[0] * nin + bias[:, :H]
    out_ref[:, H:] = b_ref[0] * nin + bias[:, H:]


_GRID = (N // MB,)
_deg_spec = pl.BlockSpec((MB, DEG_W), lambda i: (i, 0))
_half_spec_a = pl.BlockSpec((1, MB, H), lambda i: (0, i, 0))
_half_spec_b = pl.BlockSpec((1, MB, H), lambda i: (1, i, 0))
_bias_spec = pl.BlockSpec((1, D), lambda i: (0, 0))
_w_spec = pl.BlockSpec((D, D), lambda i: (0, 0))
_out2_spec = pl.BlockSpec((2, MB, H), lambda i: (0, i, 0))

_mm1 = pl.pallas_call(
    _mm1_body,
    grid=_GRID,
    in_specs=[pl.BlockSpec((MB, D), lambda i: (i, 0)), _w_spec, _deg_spec],
    out_specs=_out2_spec,
    out_shape=jax.ShapeDtypeStruct((2, N_PAD, H), jnp.float32),
)

_mm2 = pl.pallas_call(
    _mm2_body,
    grid=_GRID,
    in_specs=[_half_spec_a, _half_spec_b, _w_spec, _deg_spec, _deg_spec,
              _bias_spec],
    out_specs=_out2_spec,
    out_shape=jax.ShapeDtypeStruct((2, N_PAD, H), jnp.float32),
)

_epi = pl.pallas_call(
    _epi_body,
    grid=_GRID,
    in_specs=[_half_spec_a, _half_spec_b, _deg_spec, _bias_spec],
    out_specs=pl.BlockSpec((MB, D), lambda i: (i, 0)),
    out_shape=jax.ShapeDtypeStruct((N, D), jnp.float32),
)


def kernel(features, edge_index, W1, b1, W2, b2):
    src = edge_index[0]
    dst = edge_index[1]
    npad = E_PAD - E
    pad_idx = N + jnp.arange(npad, dtype=jnp.int32) % (N_PAD - N)
    pad_src = jnp.arange(npad, dtype=jnp.int32) % N
    srcp = jnp.concatenate([src, pad_src])
    srcp = srcp.reshape(NROWS, CHUNK)
    src2 = jnp.concatenate([srcp, srcp + N_PAD], axis=0)
    dstp = jnp.concatenate([dst, pad_idx])
    dstp = dstp.reshape(NROWS, CHUNK)
    edges2 = jnp.concatenate([
        jnp.concatenate([src, pad_idx]).reshape(NROWS, CHUNK),
        jnp.concatenate([dst, pad_idx]).reshape(NROWS, CHUNK),
    ], axis=0)

    degs = _deg(edges2)
    odeg = degs[0]
    ideg = degs[1]

    hw1 = _mm1(features, W1, odeg)
    agg1 = _agg(hw1.reshape(NC * N_PAD, H), src2, dstp).reshape(NC, N_PAD, H)
    hw2 = _mm2(agg1, agg1, W2, ideg, odeg, b1.reshape(1, D))
    agg2 = _agg(hw2.reshape(NC * N_PAD, H), src2, dstp).reshape(NC, N_PAD, H)
    return _epi(agg2, agg2, ideg, b2.reshape(1, D))


# speedup vs baseline: 1.2323x; 1.0149x over previous
import functools

import jax
import jax.numpy as jnp
from jax import lax
from jax.experimental import pallas as pl
from jax.experimental.pallas import tpu as pltpu
from jax.experimental.pallas import tpu_sc as plsc

N = 10000
E = 160000
D = 256
H = 128
NC = 2
NS = 16
CHUNK = 128
E_PAD = 163840
NROWS = E_PAD // CHUNK
NCHUNK = NROWS // NS
N_PAD = 10240
NPT = N_PAD // NS
DEG_W = 128
MB = 2000

_MESH = dict(core_axis_name="c", subcore_axis_name="s", num_cores=NC,
             num_subcores=NS)


def _deg_body(edges_hbm, out_hbm, idx_v, val_v, acc, sem):
    del sem
    c = lax.axis_index("c")
    s = lax.axis_index("s")

    def _fill(v):
        def f(j, _):
            for k in range(DEG_W // 16):
                val_v[j, pl.ds(k * 16, 16)] = jnp.full((16,), v, jnp.float32)
            return 0
        lax.fori_loop(0, CHUNK, f, 0)

    _fill(0.0)
    for k in range(NPT // CHUNK):
        pltpu.sync_copy(val_v, acc.at[pl.ds(s * NPT + k * CHUNK, CHUNK)])
    _fill(1.0)
    pltpu.sync_copy(edges_hbm.at[pl.ds(c * NROWS + s * NCHUNK, NCHUNK)], idx_v)
    plsc.subcore_barrier()

    def body(j, _):
        pltpu.sync_copy(val_v, acc.at[idx_v.at[j]], add=True)
        return 0
    lax.fori_loop(0, NCHUNK, body, 0)
    plsc.subcore_barrier()
    pltpu.sync_copy(acc.at[pl.ds(s * NPT, NPT)],
                    out_hbm.at[c].at[pl.ds(s * NPT, NPT)])


@functools.partial(
    pl.kernel,
    out_type=jax.ShapeDtypeStruct((NC, N_PAD, DEG_W), jnp.float32),
    mesh=plsc.VectorSubcoreMesh(**_MESH),
    scratch_types=[
        pltpu.VMEM((NCHUNK, CHUNK), jnp.int32),
        pltpu.VMEM((CHUNK, DEG_W), jnp.float32),
        pltpu.VMEM_SHARED((N_PAD, DEG_W), jnp.float32),
        pltpu.SemaphoreType.DMA,
    ],
)
def _deg(*args):
    _deg_body(*args)


SECS = 2
SCHUNK = NCHUNK // SECS


def _agg_body(hw_hbm, srcoff_hbm, dst_hbm, out_hbm, idx_s, idx_d,
              b0, b1, acc, g0, g1):
    c = lax.axis_index("c")
    s = lax.axis_index("s")
    bufs = (b0, b1)
    gsem = (g0, g1)

    def gather(j, b):
        pltpu.async_copy(hw_hbm.at[idx_s.at[j]], bufs[b], gsem[b])

    def gather_wait(j, b):
        pltpu.make_async_copy(hw_hbm.at[idx_s.at[j]], bufs[b],
                              gsem[b]).wait()

    base = c * NROWS + s * NCHUNK
    pltpu.sync_copy(srcoff_hbm.at[pl.ds(base, SCHUNK)], idx_s)
    pltpu.sync_copy(dst_hbm.at[pl.ds(s * NCHUNK, SCHUNK)], idx_d)
    gather(0, 0)

    def zrow(j, _):
        for k in range(H // 16):
            b1[j, pl.ds(k * 16, 16)] = jnp.zeros((16,), jnp.float32)
        return 0
    lax.fori_loop(0, CHUNK, zrow, 0)
    for k in range(NPT // CHUNK):
        pltpu.sync_copy(b1, acc.at[pl.ds(s * NPT + k * CHUNK, CHUNK)])
    plsc.subcore_barrier()

    for sec in range(SECS):
        if sec > 0:
            pltpu.sync_copy(
                srcoff_hbm.at[pl.ds(base + sec * SCHUNK, SCHUNK)], idx_s)
            pltpu.sync_copy(
                dst_hbm.at[pl.ds(s * NCHUNK + sec * SCHUNK, SCHUNK)], idx_d)
            gather(0, 0)
        gather(1, 1)

        def group(g, _):
            for b in range(2):
                j = g * 2 + b
                gather_wait(j, b)
                pltpu.sync_copy(bufs[b], acc.at[idx_d.at[j]], add=True)
                gather(j + 2, b)
            return 0
        lax.fori_loop(0, (SCHUNK - 2) // 2, group, 0)
        for j in range(SCHUNK - 2, SCHUNK):
            b = j % 2
            gather_wait(j, b)
            pltpu.sync_copy(bufs[b], acc.at[idx_d.at[j]], add=True)

    plsc.subcore_barrier()
    pltpu.sync_copy(acc.at[pl.ds(s * NPT, NPT)],
                    out_hbm.at[pl.ds(c * N_PAD + s * NPT, NPT)])


@functools.partial(
    pl.kernel,
    out_type=jax.ShapeDtypeStruct((NC * N_PAD, H), jnp.float32),
    mesh=plsc.VectorSubcoreMesh(**_MESH),
    scratch_types=[
        pltpu.VMEM((SCHUNK, CHUNK), jnp.int32),
        pltpu.VMEM((SCHUNK, CHUNK), jnp.int32),
        pltpu.VMEM((CHUNK, H), jnp.float32),
        pltpu.VMEM((CHUNK, H), jnp.float32),
        pltpu.VMEM_SHARED((N_PAD, H), jnp.float32),
        pltpu.SemaphoreType.DMA,
        pltpu.SemaphoreType.DMA,
    ],
)
def _agg(*args):
    _agg_body(*args)


def _norm(deg_col):
    return jnp.where(deg_col > 0, lax.rsqrt(deg_col), 0.0)


def _mm1_body(x_ref, w_ref, odeg_ref, out_ref):
    nout = _norm(odeg_ref[:, :1])
    hw = jnp.dot(x_ref[...], w_ref[...],
                 preferred_element_type=jnp.float32) * nout
    out_ref[0] = hw[:, :H]
    out_ref[1] = hw[:, H:]


def _mm2_body(a_ref, b_ref, w_ref, ideg_ref, odeg_ref, b1_ref, out_ref):
    nin = _norm(ideg_ref[:, :1])
    nout = _norm(odeg_ref[:, :1])
    bias = b1_ref[...]
    ha = jnp.maximum(a_ref[0] * nin + bias[:, :H], 0.0)
    hb = jnp.maximum(b_ref[0] * nin + bias[:, H:], 0.0)
    hw = (jnp.dot(ha, w_ref[:H, :], preferred_element_type=jnp.float32) +
          jnp.dot(hb, w_ref[H:, :], preferred_element_type=jnp.float32))
    hw = hw * nout
    out_ref[0] = hw[:, :H]
    out_ref[1] = hw[:, H:]


def _epi_body(a_ref, b_ref, ideg_ref, b2_ref, out_ref):
    nin = _norm(ideg_ref[:, :1])
    bias = b2_ref[...]
    out_ref[:, :H] = a_ref[0] * nin + bias[:, :H]
    out_ref[:, H:] = b_ref[0] * nin + bias[:, H:]


_GRID = (N // MB,)
_deg_spec = pl.BlockSpec((MB, DEG_W), lambda i: (i, 0))
_half_spec_a = pl.BlockSpec((1, MB, H), lambda i: (0, i, 0))
_half_spec_b = pl.BlockSpec((1, MB, H), lambda i: (1, i, 0))
_bias_spec = pl.BlockSpec((1, D), lambda i: (0, 0))
_w_spec = pl.BlockSpec((D, D), lambda i: (0, 0))
_out2_spec = pl.BlockSpec((2, MB, H), lambda i: (0, i, 0))

_mm1 = pl.pallas_call(
    _mm1_body,
    grid=_GRID,
    in_specs=[pl.BlockSpec((MB, D), lambda i: (i, 0)), _w_spec, _deg_spec],
    out_specs=_out2_spec,
    out_shape=jax.ShapeDtypeStruct((2, N_PAD, H), jnp.float32),
)

_mm2 = pl.pallas_call(
    _mm2_body,
    grid=_GRID,
    in_specs=[_half_spec_a, _half_spec_b, _w_spec, _deg_spec, _deg_spec,
              _bias_spec],
    out_specs=_out2_spec,
    out_shape=jax.ShapeDtypeStruct((2, N_PAD, H), jnp.float32),
)

_epi = pl.pallas_call(
    _epi_body,
    grid=_GRID,
    in_specs=[_half_spec_a, _half_spec_b, _deg_spec, _bias_spec],
    out_specs=pl.BlockSpec((MB, D), lambda i: (i, 0)),
    out_shape=jax.ShapeDtypeStruct((N, D), jnp.float32),
)


def kernel(features, edge_index, W1, b1, W2, b2):
    src = edge_index[0]
    dst = edge_index[1]
    npad = E_PAD - E
    pad_idx = N + jnp.arange(npad, dtype=jnp.int32) % (N_PAD - N)
    pad_src = jnp.arange(npad, dtype=jnp.int32) % N
    srcp = jnp.concatenate([src, pad_src])
    srcp = srcp.reshape(NROWS, CHUNK)
    src2 = jnp.concatenate([srcp, srcp + N_PAD], axis=0)
    dstp = jnp.concatenate([dst, pad_idx])
    dstp = dstp.reshape(NROWS, CHUNK)
    edges2 = jnp.concatenate([
        jnp.concatenate([src, pad_idx]).reshape(NROWS, CHUNK),
        jnp.concatenate([dst, pad_idx]).reshape(NROWS, CHUNK),
    ], axis=0)

    degs = _deg(edges2)
    odeg = degs[0]
    ideg = degs[1]

    hw1 = _mm1(features, W1, odeg)
    agg1 = _agg(hw1.reshape(NC * N_PAD, H), src2, dstp).reshape(NC, N_PAD, H)
    hw2 = _mm2(agg1, agg1, W2, ideg, odeg, b1.reshape(1, D))
    agg2 = _agg(hw2.reshape(NC * N_PAD, H), src2, dstp).reshape(NC, N_PAD, H)
    return _epi(agg2, agg2, ideg, b2.reshape(1, D))


# speedup vs baseline: 1.2472x; 1.0121x over previous
import functools

import jax
import jax.numpy as jnp
from jax import lax
from jax.experimental import pallas as pl
from jax.experimental.pallas import tpu as pltpu
from jax.experimental.pallas import tpu_sc as plsc

N = 10000
E = 160000
D = 256
H = 128
NC = 2
NS = 16
CHUNK = 128
E_PAD = 163840
NROWS = E_PAD // CHUNK
NCHUNK = NROWS // NS
N_PAD = 10240
NPT = N_PAD // NS
DEG_W = 128
MB = 2000

_MESH = dict(core_axis_name="c", subcore_axis_name="s", num_cores=NC,
             num_subcores=NS)


def _deg_body(edges_hbm, out_hbm, idx_v, val_v, acc, sem):
    del sem
    c = lax.axis_index("c")
    s = lax.axis_index("s")

    def _fill(v):
        def f(j, _):
            for k in range(DEG_W // 16):
                val_v[j, pl.ds(k * 16, 16)] = jnp.full((16,), v, jnp.float32)
            return 0
        lax.fori_loop(0, CHUNK, f, 0)

    _fill(0.0)
    for k in range(NPT // CHUNK):
        pltpu.sync_copy(val_v, acc.at[pl.ds(s * NPT + k * CHUNK, CHUNK)])
    _fill(1.0)
    pltpu.sync_copy(edges_hbm.at[pl.ds(c * NROWS + s * NCHUNK, NCHUNK)], idx_v)
    plsc.subcore_barrier()

    def body(j, _):
        pltpu.sync_copy(val_v, acc.at[idx_v.at[j]], add=True)
        return 0
    lax.fori_loop(0, NCHUNK, body, 0)
    plsc.subcore_barrier()
    pltpu.sync_copy(acc.at[pl.ds(s * NPT, NPT)],
                    out_hbm.at[c].at[pl.ds(s * NPT, NPT)])


@functools.partial(
    pl.kernel,
    out_type=jax.ShapeDtypeStruct((NC, N_PAD, DEG_W), jnp.float32),
    mesh=plsc.VectorSubcoreMesh(**_MESH),
    scratch_types=[
        pltpu.VMEM((NCHUNK, CHUNK), jnp.int32),
        pltpu.VMEM((CHUNK, DEG_W), jnp.float32),
        pltpu.VMEM_SHARED((N_PAD, DEG_W), jnp.float32),
        pltpu.SemaphoreType.DMA,
    ],
)
def _deg(*args):
    _deg_body(*args)


ACH = 64
ANROWS = E_PAD // ACH
ANCHUNK = ANROWS // NS
ASECS = 5
ASCHUNK = ANCHUNK // ASECS
NBUF = 4


def _agg_body(hw_hbm, srcoff_hbm, dst_hbm, out_hbm, idx_s, idx_d,
              b0, b1, b2, b3, acc, g0, g1, g2, g3):
    c = lax.axis_index("c")
    s = lax.axis_index("s")
    bufs = (b0, b1, b2, b3)
    gsem = (g0, g1, g2, g3)

    def gather(j, b):
        pltpu.async_copy(hw_hbm.at[idx_s.at[j]], bufs[b], gsem[b])

    def gather_wait(j, b):
        pltpu.make_async_copy(hw_hbm.at[idx_s.at[j]], bufs[b],
                              gsem[b]).wait()

    def scatter(j, b):
        pltpu.sync_copy(bufs[b], acc.at[idx_d.at[j]], add=True)

    base = c * ANROWS + s * ANCHUNK
    pltpu.sync_copy(srcoff_hbm.at[pl.ds(base, ASCHUNK)], idx_s)
    pltpu.sync_copy(dst_hbm.at[pl.ds(s * ANCHUNK, ASCHUNK)], idx_d)
    gather(0, 0)

    def zrow(j, _):
        for k in range(H // 16):
            b1[j, pl.ds(k * 16, 16)] = jnp.zeros((16,), jnp.float32)
        return 0
    lax.fori_loop(0, ACH, zrow, 0)
    for k in range(NPT // ACH):
        pltpu.sync_copy(b1, acc.at[pl.ds(s * NPT + k * ACH, ACH)])
    plsc.subcore_barrier()

    for sec in range(ASECS):
        if sec > 0:
            pltpu.sync_copy(
                srcoff_hbm.at[pl.ds(base + sec * ASCHUNK, ASCHUNK)], idx_s)
            pltpu.sync_copy(
                dst_hbm.at[pl.ds(s * ANCHUNK + sec * ASCHUNK, ASCHUNK)],
                idx_d)
            gather(0, 0)
        for b in range(1, NBUF):
            gather(b, b)

        def group(g, _):
            for b in range(NBUF):
                j = NBUF + g * NBUF + b
                gather_wait(j - NBUF, b)
                scatter(j - NBUF, b)
                gather(j, b)
            return 0
        lax.fori_loop(0, (ASCHUNK - NBUF) // NBUF, group, 0)
        for j in range(ASCHUNK - NBUF, ASCHUNK):
            b = j % NBUF
            gather_wait(j, b)
            scatter(j, b)

    plsc.subcore_barrier()
    pltpu.sync_copy(acc.at[pl.ds(s * NPT, NPT)],
                    out_hbm.at[pl.ds(c * N_PAD + s * NPT, NPT)])


@functools.partial(
    pl.kernel,
    out_type=jax.ShapeDtypeStruct((NC * N_PAD, H), jnp.float32),
    mesh=plsc.VectorSubcoreMesh(**_MESH),
    scratch_types=[
        pltpu.VMEM((ASCHUNK, ACH), jnp.int32),
        pltpu.VMEM((ASCHUNK, ACH), jnp.int32),
        pltpu.VMEM((ACH, H), jnp.float32),
        pltpu.VMEM((ACH, H), jnp.float32),
        pltpu.VMEM((ACH, H), jnp.float32),
        pltpu.VMEM((ACH, H), jnp.float32),
        pltpu.VMEM_SHARED((N_PAD, H), jnp.float32),
        pltpu.SemaphoreType.DMA,
        pltpu.SemaphoreType.DMA,
        pltpu.SemaphoreType.DMA,
        pltpu.SemaphoreType.DMA,
    ],
)
def _agg(*args):
    _agg_body(*args)


def _norm(deg_col):
    return jnp.where(deg_col > 0, lax.rsqrt(deg_col), 0.0)


def _mm1_body(x_ref, w_ref, odeg_ref, out_ref):
    nout = _norm(odeg_ref[:, :1])
    hw = jnp.dot(x_ref[...], w_ref[...],
                 preferred_element_type=jnp.float32) * nout
    out_ref[0] = hw[:, :H]
    out_ref[1] = hw[:, H:]


def _mm2_body(a_ref, b_ref, w_ref, ideg_ref, odeg_ref, b1_ref, out_ref):
    nin = _norm(ideg_ref[:, :1])
    nout = _norm(odeg_ref[:, :1])
    bias = b1_ref[...]
    ha = jnp.maximum(a_ref[0] * nin + bias[:, :H], 0.0)
    hb = jnp.maximum(b_ref[0] * nin + bias[:, H:], 0.0)
    hw = (jnp.dot(ha, w_ref[:H, :], preferred_element_type=jnp.float32) +
          jnp.dot(hb, w_ref[H:, :], preferred_element_type=jnp.float32))
    hw = hw * nout
    out_ref[0] = hw[:, :H]
    out_ref[1] = hw[:, H:]


def _epi_body(a_ref, b_ref, ideg_ref, b2_ref, out_ref):
    nin = _norm(ideg_ref[:, :1])
    bias = b2_ref[...]
    out_ref[:, :H] = a_ref[0] * nin + bias[:, :H]
    out_ref[:, H:] = b_ref[0] * nin + bias[:, H:]


_GRID = (N // MB,)
_deg_spec = pl.BlockSpec((MB, DEG_W), lambda i: (i, 0))
_half_spec_a = pl.BlockSpec((1, MB, H), lambda i: (0, i, 0))
_half_spec_b = pl.BlockSpec((1, MB, H), lambda i: (1, i, 0))
_bias_spec = pl.BlockSpec((1, D), lambda i: (0, 0))
_w_spec = pl.BlockSpec((D, D), lambda i: (0, 0))
_out2_spec = pl.BlockSpec((2, MB, H), lambda i: (0, i, 0))

_mm1 = pl.pallas_call(
    _mm1_body,
    grid=_GRID,
    in_specs=[pl.BlockSpec((MB, D), lambda i: (i, 0)), _w_spec, _deg_spec],
    out_specs=_out2_spec,
    out_shape=jax.ShapeDtypeStruct((2, N_PAD, H), jnp.float32),
)

_mm2 = pl.pallas_call(
    _mm2_body,
    grid=_GRID,
    in_specs=[_half_spec_a, _half_spec_b, _w_spec, _deg_spec, _deg_spec,
              _bias_spec],
    out_specs=_out2_spec,
    out_shape=jax.ShapeDtypeStruct((2, N_PAD, H), jnp.float32),
)

_epi = pl.pallas_call(
    _epi_body,
    grid=_GRID,
    in_specs=[_half_spec_a, _half_spec_b, _deg_spec, _bias_spec],
    out_specs=pl.BlockSpec((MB, D), lambda i: (i, 0)),
    out_shape=jax.ShapeDtypeStruct((N, D), jnp.float32),
)


def kernel(features, edge_index, W1, b1, W2, b2):
    src = edge_index[0]
    dst = edge_index[1]
    npad = E_PAD - E
    pad_idx = N + jnp.arange(npad, dtype=jnp.int32) % (N_PAD - N)
    pad_src = jnp.arange(npad, dtype=jnp.int32) % N
    srcp = jnp.concatenate([src, pad_src])
    srcp = srcp.reshape(ANROWS, ACH)
    src2 = jnp.concatenate([srcp, srcp + N_PAD], axis=0)
    dstp = jnp.concatenate([dst, pad_idx])
    dstp = dstp.reshape(ANROWS, ACH)
    edges2 = jnp.concatenate([
        jnp.concatenate([src, pad_idx]).reshape(NROWS, CHUNK),
        jnp.concatenate([dst, pad_idx]).reshape(NROWS, CHUNK),
    ], axis=0)

    degs = _deg(edges2)
    odeg = degs[0]
    ideg = degs[1]

    hw1 = _mm1(features, W1, odeg)
    agg1 = _agg(hw1.reshape(NC * N_PAD, H), src2, dstp).reshape(NC, N_PAD, H)
    hw2 = _mm2(agg1, agg1, W2, ideg, odeg, b1.reshape(1, D))
    agg2 = _agg(hw2.reshape(NC * N_PAD, H), src2, dstp).reshape(NC, N_PAD, H)
    return _epi(agg2, agg2, ideg, b2.reshape(1, D))
